# baseline (speedup 1.0000x reference)
import functools

import jax
import jax.numpy as jnp
from jax.experimental import pallas as pl

_ASAP_W = 0.1
_EPS = 1e-8


def _gemv_acc_body(v_ref, ev_ref, out_ref):
    @pl.when(pl.program_id(0) == 0)
    def _():
        out_ref[...] = jnp.zeros_like(out_ref)

    out_ref[...] += jax.lax.dot_general(
        v_ref[...], ev_ref[...], (((0,), (0,)), ((), ())),
        preferred_element_type=jnp.float32)


def _gemv_acc_sub_body(a_ref, b_ref, ev_ref, out_ref):
    @pl.when(pl.program_id(0) == 0)
    def _():
        out_ref[...] = jnp.zeros_like(out_ref)

    v = a_ref[...] - b_ref[...]
    out_ref[...] += jax.lax.dot_general(
        v, ev_ref[...], (((0,), (0,)), ((), ())),
        preferred_element_type=jnp.float32)


def _recon_add_body(co_ref, ec_ref, ev_ref, base_ref, out_ref):
    cc = co_ref[...] * ec_ref[...]
    out_ref[...] = base_ref[...] + jnp.sum(
        ev_ref[...] * cc, axis=1, keepdims=True)


def _recon_body(co_ref, ec_ref, ev_ref, out_ref):
    cc = co_ref[...] * ec_ref[...]
    out_ref[...] = jnp.sum(ev_ref[...] * cc, axis=1, keepdims=True)


def _edge_body(p_ref, r_ref, nn_ref, w_ref,
               pjx_ref, pjy_ref, pjz_ref, rjx_ref, rjy_ref, rjz_ref,
               gx_ref, gy_ref, gz_ref, gsum_ref, e_ref):
    px = p_ref[:, 0:1]
    py = p_ref[:, 1:2]
    pz = p_ref[:, 2:3]
    rx = r_ref[:, 0:1]
    ry = r_ref[:, 1:2]
    rz = r_ref[:, 2:3]

    ddx = px - pjx_ref[...]
    ddy = py - pjy_ref[...]
    ddz = pz - pjz_ref[...]
    drx = rx - rjx_ref[...]
    dry = ry - rjy_ref[...]
    drz = rz - rjz_ref[...]

    kio = jax.lax.broadcasted_iota(jnp.int32, w_ref.shape, 1)
    mask = (kio < nn_ref[...]).astype(jnp.float32)
    w = w_ref[...] * mask

    dddr = ddx * drx + ddy * dry + ddz * drz
    drdr = drx * drx + dry * dry + drz * drz

    num = jnp.sum(w * dddr, axis=1, keepdims=True)
    den = jnp.sum(w * drdr, axis=1, keepdims=True) + _EPS
    s = num / den
    t = s * (1.0 + _EPS / den)

    gx = 2.0 * w * (ddx - t * drx)
    gy = 2.0 * w * (ddy - t * dry)
    gz = 2.0 * w * (ddz - t * drz)
    gx_ref[...] = gx
    gy_ref[...] = gy
    gz_ref[...] = gz
    gsum_ref[...] = jnp.concatenate(
        [jnp.sum(gx, axis=1, keepdims=True),
         jnp.sum(gy, axis=1, keepdims=True),
         jnp.sum(gz, axis=1, keepdims=True)], axis=1)

    resx = ddx - s * drx
    resy = ddy - s * dry
    resz = ddz - s * drz
    e_node = jnp.sum(w * (resx * resx + resy * resy + resz * resz))

    @pl.when(pl.program_id(0) == 0)
    def _():
        e_ref[...] = jnp.zeros_like(e_ref)

    e_ref[...] += jnp.reshape(e_node, (1, 1))


@functools.partial(jax.jit, donate_argnums=())
def kernel(newSample, xyz1, xyz2, neighborsMatrix, numNeighbors, weightMatrix,
           eigC, eigV, eigVT):
    n, _ = newSample.shape
    k = neighborsMatrix.shape[1]
    nc = eigC.shape[0]
    r3 = 3 * n

    br = 3000
    bn = 2000
    g_r = r3 // br
    g_n = n // bn

    ns_flat = newSample.reshape(r3, 1)
    x2_flat = xyz2.reshape(r3, 1)
    ec2 = eigC.reshape(1, nc)

    q = pl.pallas_call(
        _gemv_acc_sub_body,
        grid=(g_r,),
        in_specs=[
            pl.BlockSpec((br, 1), lambda i: (i, 0)),
            pl.BlockSpec((br, 1), lambda i: (i, 0)),
            pl.BlockSpec((br, nc), lambda i: (i, 0)),
        ],
        out_specs=pl.BlockSpec((1, nc), lambda i: (0, 0)),
        out_shape=jax.ShapeDtypeStruct((1, nc), jnp.float32),
    )(ns_flat, x2_flat, eigV)

    p_flat = pl.pallas_call(
        _recon_add_body,
        grid=(g_r,),
        in_specs=[
            pl.BlockSpec((1, nc), lambda i: (0, 0)),
            pl.BlockSpec((1, nc), lambda i: (0, 0)),
            pl.BlockSpec((br, nc), lambda i: (i, 0)),
            pl.BlockSpec((br, 1), lambda i: (i, 0)),
        ],
        out_specs=pl.BlockSpec((br, 1), lambda i: (i, 0)),
        out_shape=jax.ShapeDtypeStruct((r3, 1), jnp.float32),
    )(q, ec2, eigV, x2_flat)

    p2 = p_flat.reshape(n, 3)

    pjx = p2[:, 0][neighborsMatrix]
    pjy = p2[:, 1][neighborsMatrix]
    pjz = p2[:, 2][neighborsMatrix]
    rjx = xyz1[:, 0][neighborsMatrix]
    rjy = xyz1[:, 1][neighborsMatrix]
    rjz = xyz1[:, 2][neighborsMatrix]
    nn2 = numNeighbors.reshape(n, 1)

    gx, gy, gz, gsum, e_acc = pl.pallas_call(
        _edge_body,
        grid=(g_n,),
        in_specs=[
            pl.BlockSpec((bn, 3), lambda i: (i, 0)),
            pl.BlockSpec((bn, 3), lambda i: (i, 0)),
            pl.BlockSpec((bn, 1), lambda i: (i, 0)),
            pl.BlockSpec((bn, k), lambda i: (i, 0)),
            pl.BlockSpec((bn, k), lambda i: (i, 0)),
            pl.BlockSpec((bn, k), lambda i: (i, 0)),
            pl.BlockSpec((bn, k), lambda i: (i, 0)),
            pl.BlockSpec((bn, k), lambda i: (i, 0)),
            pl.BlockSpec((bn, k), lambda i: (i, 0)),
            pl.BlockSpec((bn, k), lambda i: (i, 0)),
        ],
        out_specs=[
            pl.BlockSpec((bn, k), lambda i: (i, 0)),
            pl.BlockSpec((bn, k), lambda i: (i, 0)),
            pl.BlockSpec((bn, k), lambda i: (i, 0)),
            pl.BlockSpec((bn, 3), lambda i: (i, 0)),
            pl.BlockSpec((1, 1), lambda i: (0, 0)),
        ],
        out_shape=[
            jax.ShapeDtypeStruct((n, k), jnp.float32),
            jax.ShapeDtypeStruct((n, k), jnp.float32),
            jax.ShapeDtypeStruct((n, k), jnp.float32),
            jax.ShapeDtypeStruct((n, 3), jnp.float32),
            jax.ShapeDtypeStruct((1, 1), jnp.float32),
        ],
    )(p2, xyz1, nn2, weightMatrix, pjx, pjy, pjz, rjx, rjy, rjz)

    seg = neighborsMatrix.reshape(n * k)
    scx = jax.ops.segment_sum(gx.reshape(n * k), seg, num_segments=n)
    scy = jax.ops.segment_sum(gy.reshape(n * k), seg, num_segments=n)
    scz = jax.ops.segment_sum(gz.reshape(n * k), seg, num_segments=n)
    gp = gsum - jnp.stack([scx, scy, scz], axis=1)
    gp_flat = gp.reshape(r3, 1)

    t1 = pl.pallas_call(
        _gemv_acc_body,
        grid=(g_r,),
        in_specs=[
            pl.BlockSpec((br, 1), lambda i: (i, 0)),
            pl.BlockSpec((br, nc), lambda i: (i, 0)),
        ],
        out_specs=pl.BlockSpec((1, nc), lambda i: (0, 0)),
        out_shape=jax.ShapeDtypeStruct((1, nc), jnp.float32),
    )(gp_flat, eigV)

    c = _ASAP_W / n
    t2 = t1 * c

    grad_flat = pl.pallas_call(
        _recon_body,
        grid=(g_r,),
        in_specs=[
            pl.BlockSpec((1, nc), lambda i: (0, 0)),
            pl.BlockSpec((1, nc), lambda i: (0, 0)),
            pl.BlockSpec((br, nc), lambda i: (i, 0)),
        ],
        out_specs=pl.BlockSpec((br, 1), lambda i: (i, 0)),
        out_shape=jax.ShapeDtypeStruct((r3, 1), jnp.float32),
    )(t2, ec2, eigV)

    grad_new = grad_flat.reshape(n, 3)
    energy = (e_acc[0, 0] * c).astype(jnp.float32)
    return (energy, grad_new)
